# Initial kernel scaffold; baseline (speedup 1.0000x reference)
import jax
import jax.numpy as jnp
from jax.experimental import pallas as pl


def kernel(nodes_encodings, edge_index, W0, b0, W1, b1, W2, b2):
    raise NotImplementedError("write your pallas kernel here")


# speedup vs baseline: 12.3271x; 12.3271x over previous
import functools

import jax
import jax.numpy as jnp
from jax import lax
from jax.experimental import pallas as pl
from jax.experimental.pallas import tpu as pltpu
from jax.experimental.pallas import tpu_sc as plsc

_N = 10000
_NP = 10240
_D = 256
_H = 128
_E = 160000
_EP = 163840
_CHUNK = 64
_NCH = _EP // _CHUNK
_CPS = _NCH // 16
_NPAIR = _CPS // 2
_RPT = _NP // 16
_BM = 1024


def _spmm_body(hp0, hp1, srcp, dstp, zrows, o0, o1,
               src_v, dst_v, rows_v, acc_sh, gsems):
    c = lax.axis_index("c")
    s = lax.axis_index("s")
    pltpu.sync_copy(srcp.at[pl.ds(s * _NPAIR, _NPAIR)], src_v)
    pltpu.sync_copy(dstp.at[pl.ds(s * _CPS, _CPS)], dst_v)
    pltpu.sync_copy(zrows, acc_sh.at[pl.ds(s * _RPT, _RPT)])
    plsc.subcore_barrier()

    def run(hp, out):
        for b in range(2):
            pltpu.async_copy(hp.at[src_v.at[0, pl.ds(b * _CHUNK, _CHUNK)]],
                             rows_v.at[b], gsems[b])

        @pl.loop(0, _NPAIR)
        def _(p):
            for b in range(2):
                pltpu.make_async_copy(
                    hp.at[src_v.at[0, pl.ds(b * _CHUNK, _CHUNK)]],
                    rows_v.at[b], gsems[b]).wait()
                pltpu.sync_copy(rows_v.at[b], acc_sh.at[dst_v.at[2 * p + b]],
                                add=True)

                @pl.when(p + 1 < _NPAIR)
                def _():
                    pltpu.async_copy(
                        hp.at[src_v.at[p + 1, pl.ds(b * _CHUNK, _CHUNK)]],
                        rows_v.at[b], gsems[b])

        plsc.subcore_barrier()
        pltpu.sync_copy(acc_sh.at[pl.ds(s * _RPT, _RPT)],
                        out.at[pl.ds(s * _RPT, _RPT)])

    @pl.when(c == 0)
    def _():
        run(hp0, o0)

    @pl.when(c == 1)
    def _():
        run(hp1, o1)


@functools.cache
def _sc_kernels():
    mesh = plsc.VectorSubcoreMesh(core_axis_name="c", subcore_axis_name="s",
                                  num_cores=2, num_subcores=16)
    hshape = jax.ShapeDtypeStruct((_NP, _H), jnp.float32)
    spmm = pl.kernel(
        _spmm_body,
        out_type=(hshape, hshape),
        mesh=mesh,
        scratch_types=[
            pltpu.VMEM((_NPAIR, 2 * _CHUNK), jnp.int32),
            pltpu.VMEM((_CPS, _CHUNK), jnp.int32),
            pltpu.VMEM((2, _CHUNK, _H), jnp.float32),
            pltpu.VMEM_SHARED((_NP, _H), jnp.float32),
            [pltpu.SemaphoreType.DMA] * 2,
        ],
    )
    return spmm


def _dinv(p_ref):
    return lax.rsqrt(2.0 + p_ref[...])


def _l1_body(x_ref, w_ref, p_ref, o0_ref, o1_ref):
    dinv = _dinv(p_ref)
    hp = jnp.dot(x_ref[...], w_ref[...], preferred_element_type=jnp.float32)
    o0_ref[...] = hp[:, :_H] * dinv
    o1_ref[...] = hp[:, _H:] * dinv


def _mid_body(a0_ref, a1_ref, h0_ref, h1_ref, p_ref, w_ref, b_ref,
              o0_ref, o1_ref):
    dinv = _dinv(p_ref)
    z0 = jnp.maximum(dinv * (a0_ref[...] + 2.0 * h0_ref[...]) + b_ref[0:1, :], 0.0)
    z1 = jnp.maximum(dinv * (a1_ref[...] + 2.0 * h1_ref[...]) + b_ref[1:2, :], 0.0)
    x = jnp.concatenate([z0, z1], axis=1)
    hp = jnp.dot(x, w_ref[...], preferred_element_type=jnp.float32)
    o0_ref[...] = hp[:, :_H] * dinv
    o1_ref[...] = hp[:, _H:] * dinv


def _fin_body(a0_ref, a1_ref, h0_ref, h1_ref, p_ref, b_ref, o_ref):
    dinv = _dinv(p_ref)
    o_ref[:, :_H] = jnp.maximum(
        dinv * (a0_ref[...] + 2.0 * h0_ref[...]) + b_ref[0:1, :], 0.0)
    o_ref[:, _H:] = jnp.maximum(
        dinv * (a1_ref[...] + 2.0 * h1_ref[...]) + b_ref[1:2, :], 0.0)


_half_spec = pl.BlockSpec((_BM, _H), lambda i: (i, 0))
_w_spec = pl.BlockSpec((_D, _D), lambda i: (0, 0))
_b_spec = pl.BlockSpec((2, _H), lambda i: (0, 0))
_half_shape = jax.ShapeDtypeStruct((_NP, _H), jnp.float32)

_l1_tc = pl.pallas_call(
    _l1_body,
    grid=(_NP // _BM,),
    in_specs=[pl.BlockSpec((_BM, _D), lambda i: (i, 0)), _w_spec, _half_spec],
    out_specs=(_half_spec, _half_spec),
    out_shape=(_half_shape, _half_shape),
)

_mid_tc = pl.pallas_call(
    _mid_body,
    grid=(_NP // _BM,),
    in_specs=[_half_spec, _half_spec, _half_spec, _half_spec,
              _half_spec, _w_spec, _b_spec],
    out_specs=(_half_spec, _half_spec),
    out_shape=(_half_shape, _half_shape),
)

_fin_tc = pl.pallas_call(
    _fin_body,
    grid=(_NP // _BM,),
    in_specs=[_half_spec, _half_spec, _half_spec, _half_spec,
              _half_spec, _b_spec],
    out_specs=pl.BlockSpec((_BM, _D), lambda i: (i, 0)),
    out_shape=jax.ShapeDtypeStruct((_NP, _D), jnp.float32),
)


def kernel(nodes_encodings, edge_index, W0, b0, W1, b1, W2, b2):
    spmm_sc = _sc_kernels()
    x = jnp.zeros((_NP, _D), jnp.float32).at[:_N].set(nodes_encodings)
    ei = edge_index.astype(jnp.int32)
    pad = _N + (jnp.arange(_EP - _E, dtype=jnp.int32) % (_NP - _N))
    srcp = jnp.concatenate([ei[0], pad]).reshape(_NCH // 2, 2 * _CHUNK)
    dstp = jnp.concatenate([ei[1], pad]).reshape(_NCH, _CHUNK)
    zrows = jnp.zeros((_RPT, _H), jnp.float32)
    ones = jnp.ones((_NP, _H), jnp.float32)

    p, _ = spmm_sc(ones, ones, srcp, dstp, zrows)
    h0, h1 = _l1_tc(x, W0, p)
    a0, a1 = spmm_sc(h0, h1, srcp, dstp, zrows)
    h0, h1 = _mid_tc(a0, a1, h0, h1, p, W1, b0.reshape(2, _H))
    a0, a1 = spmm_sc(h0, h1, srcp, dstp, zrows)
    h0, h1 = _mid_tc(a0, a1, h0, h1, p, W2, b1.reshape(2, _H))
    a0, a1 = spmm_sc(h0, h1, srcp, dstp, zrows)
    out = _fin_tc(a0, a1, h0, h1, p, b2.reshape(2, _H))
    return out[:_N]


# speedup vs baseline: 16.2966x; 1.3220x over previous
import functools

import jax
import jax.numpy as jnp
from jax import lax
from jax.experimental import pallas as pl
from jax.experimental.pallas import tpu as pltpu
from jax.experimental.pallas import tpu_sc as plsc

_N = 10000
_NP = 10240
_D = 256
_H = 128
_E = 160000
_EP = 163840
_CHUNK = 128
_NCH = _EP // _CHUNK
_CPS = _NCH // 16
_GRP = 8
_NGRP = _CPS // _GRP
_DCPT = _NCH // 32
_RPT = _NP // 16
_BM = 1024


def _deg_body(dstp, ones_h, zrows, p0, p1, dst_v, ones_v, acc_sh, ssem):
    c = lax.axis_index("c")
    s = lax.axis_index("s")
    base = (c * 16 + s) * _DCPT
    pltpu.sync_copy(dstp.at[pl.ds(base, _DCPT)], dst_v)
    pltpu.sync_copy(ones_h, ones_v)
    pltpu.sync_copy(zrows, acc_sh.at[pl.ds(s * _RPT, _RPT)])
    plsc.subcore_barrier()

    @pl.loop(0, _DCPT, step=8)
    def _(j):
        for k in range(8):
            pltpu.async_copy(ones_v, acc_sh.at[dst_v.at[j + k]], ssem,
                             add=True)
        for k in range(8):
            pltpu.make_async_copy(ones_v, acc_sh.at[dst_v.at[j + k]],
                                  ssem).wait()

    plsc.subcore_barrier()

    @pl.when(c == 0)
    def _():
        pltpu.sync_copy(acc_sh.at[pl.ds(s * _RPT, _RPT)],
                        p0.at[pl.ds(s * _RPT, _RPT)])

    @pl.when(c == 1)
    def _():
        pltpu.sync_copy(acc_sh.at[pl.ds(s * _RPT, _RPT)],
                        p1.at[pl.ds(s * _RPT, _RPT)])


def _spmm_body(hp0, hp1, srcp, dstp, zrows, o0, o1,
               src_v, dst_g, rows_v, acc_sh, gsems, dsem):
    c = lax.axis_index("c")
    s = lax.axis_index("s")
    pltpu.sync_copy(srcp.at[pl.ds(s * _CPS, _CPS)], src_v)
    pltpu.sync_copy(zrows, acc_sh.at[pl.ds(s * _RPT, _RPT)])
    plsc.subcore_barrier()

    def run(hp, out):
        for b in range(2):
            pltpu.async_copy(hp.at[src_v.at[b]], rows_v.at[b], gsems[b])
        pltpu.sync_copy(dstp.at[pl.ds(s * _CPS, _GRP)], dst_g[0])

        @pl.loop(0, _NGRP, step=2)
        def _(g):
            for gb in range(2):
                gg = g + gb

                @pl.when(gg + 1 < _NGRP)
                def _():
                    pltpu.async_copy(
                        dstp.at[pl.ds(s * _CPS + (gg + 1) * _GRP, _GRP)],
                        dst_g[1 - gb], dsem)

                for k in range(_GRP):
                    b = k % 2
                    jj = gg * _GRP + k
                    pltpu.make_async_copy(hp.at[src_v.at[b]], rows_v.at[b],
                                          gsems[b]).wait()
                    pltpu.sync_copy(rows_v.at[b], acc_sh.at[dst_g[gb].at[k]],
                                    add=True)

                    @pl.when(jj + 2 < _CPS)
                    def _():
                        pltpu.async_copy(hp.at[src_v.at[jj + 2]], rows_v.at[b],
                                         gsems[b])

                @pl.when(gg + 1 < _NGRP)
                def _():
                    pltpu.make_async_copy(
                        dstp.at[pl.ds(s * _CPS, _GRP)], dst_g[1 - gb],
                        dsem).wait()

        plsc.subcore_barrier()
        pltpu.sync_copy(acc_sh.at[pl.ds(s * _RPT, _RPT)],
                        out.at[pl.ds(s * _RPT, _RPT)])

    @pl.when(c == 0)
    def _():
        run(hp0, o0)

    @pl.when(c == 1)
    def _():
        run(hp1, o1)


@functools.cache
def _sc_kernels():
    mesh = plsc.VectorSubcoreMesh(core_axis_name="c", subcore_axis_name="s",
                                  num_cores=2, num_subcores=16)
    hshape = jax.ShapeDtypeStruct((_NP, _H), jnp.float32)
    deg = pl.kernel(
        _deg_body,
        out_type=(hshape, hshape),
        mesh=mesh,
        scratch_types=[
            pltpu.VMEM((_DCPT, _CHUNK), jnp.int32),
            pltpu.VMEM((_CHUNK, _H), jnp.float32),
            pltpu.VMEM_SHARED((_NP, _H), jnp.float32),
            pltpu.SemaphoreType.DMA,
        ],
    )
    spmm = pl.kernel(
        _spmm_body,
        out_type=(hshape, hshape),
        mesh=mesh,
        scratch_types=[
            pltpu.VMEM((_CPS, _CHUNK), jnp.int32),
            [pltpu.VMEM((_GRP, _CHUNK), jnp.int32)] * 2,
            pltpu.VMEM((2, _CHUNK, _H), jnp.float32),
            pltpu.VMEM_SHARED((_NP, _H), jnp.float32),
            [pltpu.SemaphoreType.DMA] * 2,
            pltpu.SemaphoreType.DMA,
        ],
    )
    return deg, spmm


def _dinv(p0_ref, p1_ref):
    return lax.rsqrt(2.0 + p0_ref[...] + p1_ref[...])


def _l1_body(x_ref, w_ref, p0_ref, p1_ref, o0_ref, o1_ref):
    dinv = _dinv(p0_ref, p1_ref)
    hp = jnp.dot(x_ref[...], w_ref[...], preferred_element_type=jnp.float32)
    o0_ref[...] = hp[:, :_H] * dinv
    o1_ref[...] = hp[:, _H:] * dinv


def _mid_body(a0_ref, a1_ref, h0_ref, h1_ref, p0_ref, p1_ref, w_ref, b_ref,
              o0_ref, o1_ref):
    dinv = _dinv(p0_ref, p1_ref)
    z0 = jnp.maximum(dinv * (a0_ref[...] + 2.0 * h0_ref[...]) + b_ref[0:1, :], 0.0)
    z1 = jnp.maximum(dinv * (a1_ref[...] + 2.0 * h1_ref[...]) + b_ref[1:2, :], 0.0)
    x = jnp.concatenate([z0, z1], axis=1)
    hp = jnp.dot(x, w_ref[...], preferred_element_type=jnp.float32)
    o0_ref[...] = hp[:, :_H] * dinv
    o1_ref[...] = hp[:, _H:] * dinv


def _fin_body(a0_ref, a1_ref, h0_ref, h1_ref, p0_ref, p1_ref, b_ref, o_ref):
    dinv = _dinv(p0_ref, p1_ref)
    o_ref[:, :_H] = jnp.maximum(
        dinv * (a0_ref[...] + 2.0 * h0_ref[...]) + b_ref[0:1, :], 0.0)
    o_ref[:, _H:] = jnp.maximum(
        dinv * (a1_ref[...] + 2.0 * h1_ref[...]) + b_ref[1:2, :], 0.0)


_half_spec = pl.BlockSpec((_BM, _H), lambda i: (i, 0))
_w_spec = pl.BlockSpec((_D, _D), lambda i: (0, 0))
_b_spec = pl.BlockSpec((2, _H), lambda i: (0, 0))
_half_shape = jax.ShapeDtypeStruct((_NP, _H), jnp.float32)

_l1_tc = pl.pallas_call(
    _l1_body,
    grid=(_NP // _BM,),
    in_specs=[pl.BlockSpec((_BM, _D), lambda i: (i, 0)), _w_spec,
              _half_spec, _half_spec],
    out_specs=(_half_spec, _half_spec),
    out_shape=(_half_shape, _half_shape),
)

_mid_tc = pl.pallas_call(
    _mid_body,
    grid=(_NP // _BM,),
    in_specs=[_half_spec, _half_spec, _half_spec, _half_spec,
              _half_spec, _half_spec, _w_spec, _b_spec],
    out_specs=(_half_spec, _half_spec),
    out_shape=(_half_shape, _half_shape),
)

_fin_tc = pl.pallas_call(
    _fin_body,
    grid=(_NP // _BM,),
    in_specs=[_half_spec, _half_spec, _half_spec, _half_spec,
              _half_spec, _half_spec, _b_spec],
    out_specs=pl.BlockSpec((_BM, _D), lambda i: (i, 0)),
    out_shape=jax.ShapeDtypeStruct((_NP, _D), jnp.float32),
)


def kernel(nodes_encodings, edge_index, W0, b0, W1, b1, W2, b2):
    deg_sc, spmm_sc = _sc_kernels()
    x = jnp.zeros((_NP, _D), jnp.float32).at[:_N].set(nodes_encodings)
    ei = edge_index.astype(jnp.int32)
    pad = _N + (jnp.arange(_EP - _E, dtype=jnp.int32) % (_NP - _N))
    srcp = jnp.concatenate([ei[0], pad]).reshape(_NCH, _CHUNK)
    dstp = jnp.concatenate([ei[1], pad]).reshape(_NCH, _CHUNK)
    zrows = jnp.zeros((_RPT, _H), jnp.float32)
    ones_h = jnp.ones((_CHUNK, _H), jnp.float32)

    p0, p1 = deg_sc(dstp, ones_h, zrows)
    h0, h1 = _l1_tc(x, W0, p0, p1)
    a0, a1 = spmm_sc(h0, h1, srcp, dstp, zrows)
    h0, h1 = _mid_tc(a0, a1, h0, h1, p0, p1, W1, b0.reshape(2, _H))
    a0, a1 = spmm_sc(h0, h1, srcp, dstp, zrows)
    h0, h1 = _mid_tc(a0, a1, h0, h1, p0, p1, W2, b1.reshape(2, _H))
    a0, a1 = spmm_sc(h0, h1, srcp, dstp, zrows)
    out = _fin_tc(a0, a1, h0, h1, p0, p1, b2.reshape(2, _H))
    return out[:_N]


# speedup vs baseline: 17.2778x; 1.0602x over previous
import functools

import jax
import jax.numpy as jnp
from jax import lax
from jax.experimental import pallas as pl
from jax.experimental.pallas import tpu as pltpu
from jax.experimental.pallas import tpu_sc as plsc

_N = 10000
_NP = 10240
_D = 256
_H = 128
_E = 160000
_EP = 163840
_CHUNK = 128
_NCH = _EP // _CHUNK
_CPS = _NCH // 16
_GRP = 8
_NGRP = _CPS // _GRP
_DCPT = _NCH // 32
_RPT = _NP // 16
_BM = 1024


def _splat(v, i):
    return lax.gather(
        v, jnp.full((16, 1), i, jnp.int32),
        lax.GatherDimensionNumbers(offset_dims=(), collapsed_slice_dims=(0,),
                                   start_index_map=(0,)),
        slice_sizes=(1,), mode=lax.GatherScatterMode.PROMISE_IN_BOUNDS)


def _deg_body(dstp, zcol, p0, p1, dst_v, degl, part_v, rep_v, sh):
    c = lax.axis_index("c")
    s = lax.axis_index("s")
    base = (c * 16 + s) * _DCPT
    pltpu.sync_copy(dstp.at[pl.ds(base, _DCPT)], dst_v)
    pltpu.sync_copy(zcol, degl)
    ones16 = jnp.full((16,), 1.0, jnp.float32)

    @pl.loop(0, _DCPT)
    def _(j):
        for q in range(_CHUNK // 16):
            iv = dst_v[j, pl.ds(q * 16, 16)]
            plsc.addupdate_scatter(degl, [iv], ones16)

    for seg in range(16):
        pltpu.sync_copy(degl.at[pl.ds(seg * _RPT, _RPT)], sh.at[s, seg])
    plsc.subcore_barrier()

    for t in range(16):
        pltpu.sync_copy(sh.at[t, s], part_v.at[t])

    def emit(p):
        @pl.loop(0, _RPT // 16)
        def _(g):
            v = part_v[0, pl.ds(g * 16, 16)]
            for t in range(1, 16):
                v = v + part_v[t, pl.ds(g * 16, 16)]
            for i in range(16):
                row = _splat(v, i)
                for l in range(_H // 16):
                    rep_v[i, pl.ds(l * 16, 16)] = row
            pltpu.sync_copy(rep_v, p.at[pl.ds(s * _RPT + g * 16, 16)])

    @pl.when(c == 0)
    def _():
        emit(p0)

    @pl.when(c == 1)
    def _():
        emit(p1)


def _spmm_body(hp0, hp1, srcp, dstp, zrows, o0, o1,
               src_v, dst_g, rows_v, acc_sh, gsems, dsem):
    c = lax.axis_index("c")
    s = lax.axis_index("s")
    pltpu.sync_copy(srcp.at[pl.ds(s * _CPS, _CPS)], src_v)
    pltpu.sync_copy(zrows, acc_sh.at[pl.ds(s * _RPT, _RPT)])

    def run(hp, out):
        for b in range(2):
            pltpu.async_copy(hp.at[src_v.at[b]], rows_v.at[b], gsems[b])
        pltpu.sync_copy(dstp.at[pl.ds(s * _CPS, _GRP)], dst_g[0])
        plsc.subcore_barrier()

        @pl.loop(0, _NGRP, step=2)
        def _(g):
            for gb in range(2):
                gg = g + gb

                @pl.when(gg + 1 < _NGRP)
                def _():
                    pltpu.async_copy(
                        dstp.at[pl.ds(s * _CPS + (gg + 1) * _GRP, _GRP)],
                        dst_g[1 - gb], dsem)

                for k in range(_GRP):
                    b = k % 2
                    jj = gg * _GRP + k
                    pltpu.make_async_copy(hp.at[src_v.at[b]], rows_v.at[b],
                                          gsems[b]).wait()
                    pltpu.sync_copy(rows_v.at[b], acc_sh.at[dst_g[gb].at[k]],
                                    add=True)

                    @pl.when(jj + 2 < _CPS)
                    def _():
                        pltpu.async_copy(hp.at[src_v.at[jj + 2]], rows_v.at[b],
                                         gsems[b])

                @pl.when(gg + 1 < _NGRP)
                def _():
                    pltpu.make_async_copy(
                        dstp.at[pl.ds(s * _CPS, _GRP)], dst_g[1 - gb],
                        dsem).wait()

        plsc.subcore_barrier()
        pltpu.sync_copy(acc_sh.at[pl.ds(s * _RPT, _RPT)],
                        out.at[pl.ds(s * _RPT, _RPT)])

    @pl.when(c == 0)
    def _():
        run(hp0, o0)

    @pl.when(c == 1)
    def _():
        run(hp1, o1)


@functools.cache
def _sc_kernels():
    mesh = plsc.VectorSubcoreMesh(core_axis_name="c", subcore_axis_name="s",
                                  num_cores=2, num_subcores=16)
    hshape = jax.ShapeDtypeStruct((_NP, _H), jnp.float32)
    deg = pl.kernel(
        _deg_body,
        out_type=(hshape, hshape),
        mesh=mesh,
        scratch_types=[
            pltpu.VMEM((_DCPT, _CHUNK), jnp.int32),
            pltpu.VMEM((_NP,), jnp.float32),
            pltpu.VMEM((16, _RPT), jnp.float32),
            pltpu.VMEM((16, _H), jnp.float32),
            pltpu.VMEM_SHARED((16, 16, _RPT), jnp.float32),
        ],
        compiler_params=pltpu.CompilerParams(needs_layout_passes=False),
    )
    spmm = pl.kernel(
        _spmm_body,
        out_type=(hshape, hshape),
        mesh=mesh,
        scratch_types=[
            pltpu.VMEM((_CPS, _CHUNK), jnp.int32),
            [pltpu.VMEM((_GRP, _CHUNK), jnp.int32)] * 2,
            pltpu.VMEM((2, _CHUNK, _H), jnp.float32),
            pltpu.VMEM_SHARED((_NP, _H), jnp.float32),
            [pltpu.SemaphoreType.DMA] * 2,
            pltpu.SemaphoreType.DMA,
        ],
    )
    return deg, spmm


def _dinv(p0_ref, p1_ref):
    return lax.rsqrt(2.0 + p0_ref[...] + p1_ref[...])


def _l1_body(x_ref, w_ref, p0_ref, p1_ref, o0_ref, o1_ref):
    dinv = _dinv(p0_ref, p1_ref)
    hp = jnp.dot(x_ref[...], w_ref[...], preferred_element_type=jnp.float32)
    o0_ref[...] = hp[:, :_H] * dinv
    o1_ref[...] = hp[:, _H:] * dinv


def _mid_body(a0_ref, a1_ref, h0_ref, h1_ref, p0_ref, p1_ref, w_ref, b_ref,
              o0_ref, o1_ref):
    dinv = _dinv(p0_ref, p1_ref)
    z0 = jnp.maximum(dinv * (a0_ref[...] + 2.0 * h0_ref[...]) + b_ref[0:1, :], 0.0)
    z1 = jnp.maximum(dinv * (a1_ref[...] + 2.0 * h1_ref[...]) + b_ref[1:2, :], 0.0)
    x = jnp.concatenate([z0, z1], axis=1)
    hp = jnp.dot(x, w_ref[...], preferred_element_type=jnp.float32)
    o0_ref[...] = hp[:, :_H] * dinv
    o1_ref[...] = hp[:, _H:] * dinv


def _fin_body(a0_ref, a1_ref, h0_ref, h1_ref, p0_ref, p1_ref, b_ref, o_ref):
    dinv = _dinv(p0_ref, p1_ref)
    o_ref[:, :_H] = jnp.maximum(
        dinv * (a0_ref[...] + 2.0 * h0_ref[...]) + b_ref[0:1, :], 0.0)
    o_ref[:, _H:] = jnp.maximum(
        dinv * (a1_ref[...] + 2.0 * h1_ref[...]) + b_ref[1:2, :], 0.0)


_half_spec = pl.BlockSpec((_BM, _H), lambda i: (i, 0))
_w_spec = pl.BlockSpec((_D, _D), lambda i: (0, 0))
_b_spec = pl.BlockSpec((2, _H), lambda i: (0, 0))
_half_shape = jax.ShapeDtypeStruct((_NP, _H), jnp.float32)

_l1_tc = pl.pallas_call(
    _l1_body,
    grid=(_NP // _BM,),
    in_specs=[pl.BlockSpec((_BM, _D), lambda i: (i, 0)), _w_spec,
              _half_spec, _half_spec],
    out_specs=(_half_spec, _half_spec),
    out_shape=(_half_shape, _half_shape),
)

_mid_tc = pl.pallas_call(
    _mid_body,
    grid=(_NP // _BM,),
    in_specs=[_half_spec, _half_spec, _half_spec, _half_spec,
              _half_spec, _half_spec, _w_spec, _b_spec],
    out_specs=(_half_spec, _half_spec),
    out_shape=(_half_shape, _half_shape),
)

_fin_tc = pl.pallas_call(
    _fin_body,
    grid=(_NP // _BM,),
    in_specs=[_half_spec, _half_spec, _half_spec, _half_spec,
              _half_spec, _half_spec, _b_spec],
    out_specs=pl.BlockSpec((_BM, _D), lambda i: (i, 0)),
    out_shape=jax.ShapeDtypeStruct((_NP, _D), jnp.float32),
)


def kernel(nodes_encodings, edge_index, W0, b0, W1, b1, W2, b2):
    deg_sc, spmm_sc = _sc_kernels()
    x = jnp.zeros((_NP, _D), jnp.float32).at[:_N].set(nodes_encodings)
    ei = edge_index.astype(jnp.int32)
    pad = _N + (jnp.arange(_EP - _E, dtype=jnp.int32) % (_NP - _N))
    srcp = jnp.concatenate([ei[0], pad]).reshape(_NCH, _CHUNK)
    dstp = jnp.concatenate([ei[1], pad]).reshape(_NCH, _CHUNK)
    zrows = jnp.zeros((_RPT, _H), jnp.float32)
    zcol = jnp.zeros((_NP,), jnp.float32)

    p0, p1 = deg_sc(dstp, zcol)
    h0, h1 = _l1_tc(x, W0, p0, p1)
    a0, a1 = spmm_sc(h0, h1, srcp, dstp, zrows)
    h0, h1 = _mid_tc(a0, a1, h0, h1, p0, p1, W1, b0.reshape(2, _H))
    a0, a1 = spmm_sc(h0, h1, srcp, dstp, zrows)
    h0, h1 = _mid_tc(a0, a1, h0, h1, p0, p1, W2, b1.reshape(2, _H))
    a0, a1 = spmm_sc(h0, h1, srcp, dstp, zrows)
    out = _fin_tc(a0, a1, h0, h1, p0, p1, b2.reshape(2, _H))
    return out[:_N]


# speedup vs baseline: 17.7295x; 1.0261x over previous
import functools

import jax
import jax.numpy as jnp
from jax import lax
from jax.experimental import pallas as pl
from jax.experimental.pallas import tpu as pltpu
from jax.experimental.pallas import tpu_sc as plsc

_N = 10000
_NP = 10240
_D = 256
_H = 128
_E = 160000
_EP = 163840
_CHUNK = 128
_NCH = _EP // _CHUNK
_CPS = _NCH // 16
_GRP = 8
_NGRP = _CPS // _GRP
_DCPT = _NCH // 16
_RPT = _NP // 16
_BM = 1024


def _splat(v, i):
    return lax.gather(
        v, jnp.full((16, 1), i, jnp.int32),
        lax.GatherDimensionNumbers(offset_dims=(), collapsed_slice_dims=(0,),
                                   start_index_map=(0,)),
        slice_sizes=(1,), mode=lax.GatherScatterMode.PROMISE_IN_BOUNDS)


def _deg_body(dstp, zcol, p0, dst_v, degl, part_v, rep_v, sh):
    c = lax.axis_index("c")
    s = lax.axis_index("s")
    base = s * _DCPT
    pltpu.sync_copy(dstp.at[pl.ds(base, _DCPT)], dst_v)
    pltpu.sync_copy(zcol, degl)
    ones16 = jnp.full((16,), 1.0, jnp.float32)

    @pl.loop(0, _DCPT)
    def _(j):
        for q in range(_CHUNK // 16):
            iv = dst_v[j, pl.ds(q * 16, 16)]
            plsc.addupdate_scatter(degl, [iv], ones16)

    for seg in range(16):
        pltpu.sync_copy(degl.at[pl.ds(seg * _RPT, _RPT)], sh.at[s, seg])
    plsc.subcore_barrier()

    for t in range(16):
        pltpu.sync_copy(sh.at[t, s], part_v.at[t])

    def emit(p):
        @pl.loop(0, _RPT // 16)
        def _(g):
            v = part_v[0, pl.ds(g * 16, 16)]
            for t in range(1, 16):
                v = v + part_v[t, pl.ds(g * 16, 16)]
            for i in range(16):
                row = _splat(v, i)
                for l in range(_H // 16):
                    rep_v[i, pl.ds(l * 16, 16)] = row
            pltpu.sync_copy(rep_v, p.at[pl.ds(s * _RPT + g * 16, 16)])

    @pl.when(c == 0)
    def _():
        emit(p0)


def _spmm_body(hp0, hp1, srcp, dstp, o0, o1,
               src_v, dst_g, rows_v, acc_sh, gsems, dsem):
    c = lax.axis_index("c")
    s = lax.axis_index("s")
    pltpu.sync_copy(srcp.at[pl.ds(s * _CPS, _CPS)], src_v)

    z16 = jnp.zeros((16,), jnp.float32)

    @pl.loop(0, _CHUNK)
    def _(r):
        for l in range(_H // 16):
            rows_v[0, r, pl.ds(l * 16, 16)] = z16

    for q in range(_RPT // _CHUNK):
        pltpu.sync_copy(rows_v.at[0],
                        acc_sh.at[pl.ds(s * _RPT + q * _CHUNK, _CHUNK)])

    def run(hp, out):
        for b in range(2):
            pltpu.async_copy(hp.at[src_v.at[b]], rows_v.at[b], gsems[b])
        pltpu.sync_copy(dstp.at[pl.ds(s * _CPS, _GRP)], dst_g[0])
        plsc.subcore_barrier()

        @pl.loop(0, _NGRP, step=2)
        def _(g):
            for gb in range(2):
                gg = g + gb

                @pl.when(gg + 1 < _NGRP)
                def _():
                    pltpu.async_copy(
                        dstp.at[pl.ds(s * _CPS + (gg + 1) * _GRP, _GRP)],
                        dst_g[1 - gb], dsem)

                for k in range(_GRP):
                    b = k % 2
                    jj = gg * _GRP + k
                    pltpu.make_async_copy(hp.at[src_v.at[b]], rows_v.at[b],
                                          gsems[b]).wait()
                    pltpu.sync_copy(rows_v.at[b], acc_sh.at[dst_g[gb].at[k]],
                                    add=True)

                    @pl.when(jj + 2 < _CPS)
                    def _():
                        pltpu.async_copy(hp.at[src_v.at[jj + 2]], rows_v.at[b],
                                         gsems[b])

                @pl.when(gg + 1 < _NGRP)
                def _():
                    pltpu.make_async_copy(
                        dstp.at[pl.ds(s * _CPS, _GRP)], dst_g[1 - gb],
                        dsem).wait()

        plsc.subcore_barrier()
        pltpu.sync_copy(acc_sh.at[pl.ds(s * _RPT, _RPT)],
                        out.at[pl.ds(s * _RPT, _RPT)])

    @pl.when(c == 0)
    def _():
        run(hp0, o0)

    @pl.when(c == 1)
    def _():
        run(hp1, o1)


@functools.cache
def _sc_kernels():
    mesh = plsc.VectorSubcoreMesh(core_axis_name="c", subcore_axis_name="s",
                                  num_cores=2, num_subcores=16)
    hshape = jax.ShapeDtypeStruct((_NP, _H), jnp.float32)
    deg = pl.kernel(
        _deg_body,
        out_type=hshape,
        mesh=mesh,
        scratch_types=[
            pltpu.VMEM((_DCPT, _CHUNK), jnp.int32),
            pltpu.VMEM((_NP,), jnp.float32),
            pltpu.VMEM((16, _RPT), jnp.float32),
            pltpu.VMEM((16, _H), jnp.float32),
            pltpu.VMEM_SHARED((16, 16, _RPT), jnp.float32),
        ],
        compiler_params=pltpu.CompilerParams(needs_layout_passes=False),
    )
    spmm = pl.kernel(
        _spmm_body,
        out_type=(hshape, hshape),
        mesh=mesh,
        scratch_types=[
            pltpu.VMEM((_CPS, _CHUNK), jnp.int32),
            [pltpu.VMEM((_GRP, _CHUNK), jnp.int32)] * 2,
            pltpu.VMEM((2, _CHUNK, _H), jnp.float32),
            pltpu.VMEM_SHARED((_NP, _H), jnp.float32),
            [pltpu.SemaphoreType.DMA] * 2,
            pltpu.SemaphoreType.DMA,
        ],
        compiler_params=pltpu.CompilerParams(needs_layout_passes=False),
    )
    return deg, spmm


def _dinv(p_ref):
    return lax.rsqrt(2.0 + p_ref[...])


def _l1_body(x_ref, w_ref, p_ref, o0_ref, o1_ref):
    dinv = _dinv(p_ref)
    hp = jnp.dot(x_ref[...], w_ref[...], preferred_element_type=jnp.float32)
    o0_ref[...] = hp[:, :_H] * dinv
    o1_ref[...] = hp[:, _H:] * dinv


def _mid_body(a0_ref, a1_ref, h0_ref, h1_ref, p_ref, w_ref, b_ref,
              o0_ref, o1_ref):
    dinv = _dinv(p_ref)
    z0 = jnp.maximum(dinv * (a0_ref[...] + 2.0 * h0_ref[...]) + b_ref[0:1, :], 0.0)
    z1 = jnp.maximum(dinv * (a1_ref[...] + 2.0 * h1_ref[...]) + b_ref[1:2, :], 0.0)
    x = jnp.concatenate([z0, z1], axis=1)
    hp = jnp.dot(x, w_ref[...], preferred_element_type=jnp.float32)
    o0_ref[...] = hp[:, :_H] * dinv
    o1_ref[...] = hp[:, _H:] * dinv


def _fin_body(a0_ref, a1_ref, h0_ref, h1_ref, p_ref, b_ref, o_ref):
    dinv = _dinv(p_ref)
    o_ref[:, :_H] = jnp.maximum(
        dinv * (a0_ref[...] + 2.0 * h0_ref[...]) + b_ref[0:1, :], 0.0)
    o_ref[:, _H:] = jnp.maximum(
        dinv * (a1_ref[...] + 2.0 * h1_ref[...]) + b_ref[1:2, :], 0.0)


_half_spec = pl.BlockSpec((_BM, _H), lambda i: (i, 0))
_w_spec = pl.BlockSpec((_D, _D), lambda i: (0, 0))
_b_spec = pl.BlockSpec((2, _H), lambda i: (0, 0))
_half_shape = jax.ShapeDtypeStruct((_NP, _H), jnp.float32)

_l1_tc = pl.pallas_call(
    _l1_body,
    grid=(_NP // _BM,),
    in_specs=[pl.BlockSpec((_BM, _D), lambda i: (i, 0)), _w_spec, _half_spec],
    out_specs=(_half_spec, _half_spec),
    out_shape=(_half_shape, _half_shape),
)

_mid_tc = pl.pallas_call(
    _mid_body,
    grid=(_NP // _BM,),
    in_specs=[_half_spec, _half_spec, _half_spec, _half_spec,
              _half_spec, _w_spec, _b_spec],
    out_specs=(_half_spec, _half_spec),
    out_shape=(_half_shape, _half_shape),
)

_fin_tc = pl.pallas_call(
    _fin_body,
    grid=(_NP // _BM,),
    in_specs=[_half_spec, _half_spec, _half_spec, _half_spec,
              _half_spec, _b_spec],
    out_specs=pl.BlockSpec((_BM, _D), lambda i: (i, 0)),
    out_shape=jax.ShapeDtypeStruct((_NP, _D), jnp.float32),
)


def kernel(nodes_encodings, edge_index, W0, b0, W1, b1, W2, b2):
    deg_sc, spmm_sc = _sc_kernels()
    x = jnp.zeros((_NP, _D), jnp.float32).at[:_N].set(nodes_encodings)
    ei = edge_index.astype(jnp.int32)
    pad = _N + (jnp.arange(_EP - _E, dtype=jnp.int32) % (_NP - _N))
    srcp = jnp.concatenate([ei[0], pad]).reshape(_NCH, _CHUNK)
    dstp = jnp.concatenate([ei[1], pad]).reshape(_NCH, _CHUNK)
    zcol = jnp.zeros((_NP,), jnp.float32)

    p = deg_sc(dstp, zcol)
    h0, h1 = _l1_tc(x, W0, p)
    a0, a1 = spmm_sc(h0, h1, srcp, dstp)
    h0, h1 = _mid_tc(a0, a1, h0, h1, p, W1, b0.reshape(2, _H))
    a0, a1 = spmm_sc(h0, h1, srcp, dstp)
    h0, h1 = _mid_tc(a0, a1, h0, h1, p, W2, b1.reshape(2, _H))
    a0, a1 = spmm_sc(h0, h1, srcp, dstp)
    out = _fin_tc(a0, a1, h0, h1, p, b2.reshape(2, _H))
    return out[:_N]


# speedup vs baseline: 18.0162x; 1.0162x over previous
import functools

import jax
import jax.numpy as jnp
from jax import lax
from jax.experimental import pallas as pl
from jax.experimental.pallas import tpu as pltpu
from jax.experimental.pallas import tpu_sc as plsc

_N = 10000
_NP = 10240
_D = 256
_H = 128
_E = 160000
_EP = 163840
_CHUNK = 128
_NCH = _EP // _CHUNK
_CPS = _NCH // 16
_GRP = 8
_NGRP = _CPS // _GRP
_DCPT = _NCH // 16
_RPT = _NP // 16
_BM = 2048


def _splat(v, i):
    return lax.gather(
        v, jnp.full((16, 1), i, jnp.int32),
        lax.GatherDimensionNumbers(offset_dims=(), collapsed_slice_dims=(0,),
                                   start_index_map=(0,)),
        slice_sizes=(1,), mode=lax.GatherScatterMode.PROMISE_IN_BOUNDS)


def _deg_body(dstp, zcol, p0, dst_v, degl, part_v, rep_v, sh):
    c = lax.axis_index("c")
    s = lax.axis_index("s")
    base = s * _DCPT
    pltpu.sync_copy(dstp.at[pl.ds(base, _DCPT)], dst_v)
    pltpu.sync_copy(zcol, degl)
    ones16 = jnp.full((16,), 1.0, jnp.float32)

    @pl.loop(0, _DCPT)
    def _(j):
        for q in range(_CHUNK // 16):
            iv = dst_v[j, pl.ds(q * 16, 16)]
            plsc.addupdate_scatter(degl, [iv], ones16)

    for seg in range(16):
        pltpu.sync_copy(degl.at[pl.ds(seg * _RPT, _RPT)], sh.at[s, seg])
    plsc.subcore_barrier()

    for t in range(16):
        pltpu.sync_copy(sh.at[t, s], part_v.at[t])

    def emit(p):
        @pl.loop(0, _RPT // 16)
        def _(g):
            v = part_v[0, pl.ds(g * 16, 16)]
            for t in range(1, 16):
                v = v + part_v[t, pl.ds(g * 16, 16)]
            for i in range(16):
                row = _splat(v, i)
                for l in range(_H // 16):
                    rep_v[i, pl.ds(l * 16, 16)] = row
            pltpu.sync_copy(rep_v, p.at[pl.ds(s * _RPT + g * 16, 16)])

    @pl.when(c == 0)
    def _():
        emit(p0)


def _spmm_body(hp0, hp1, srcp, dstp, o0, o1,
               src_v, dst_g, rows_v, acc_sh, gsems, dsem):
    c = lax.axis_index("c")
    s = lax.axis_index("s")
    pltpu.sync_copy(srcp.at[pl.ds(s * _CPS, _CPS)], src_v)

    z16 = jnp.zeros((16,), jnp.float32)

    @pl.loop(0, _CHUNK)
    def _(r):
        for l in range(_H // 16):
            rows_v[0, r, pl.ds(l * 16, 16)] = z16

    for q in range(_RPT // _CHUNK):
        pltpu.sync_copy(rows_v.at[0],
                        acc_sh.at[pl.ds(s * _RPT + q * _CHUNK, _CHUNK)])

    def run(hp, out):
        for b in range(2):
            pltpu.async_copy(hp.at[src_v.at[b]], rows_v.at[b], gsems[b])
        pltpu.sync_copy(dstp.at[pl.ds(s * _CPS, _GRP)], dst_g[0])
        plsc.subcore_barrier()

        @pl.loop(0, _NGRP, step=2)
        def _(g):
            for gb in range(2):
                gg = g + gb

                @pl.when(gg + 1 < _NGRP)
                def _():
                    pltpu.async_copy(
                        dstp.at[pl.ds(s * _CPS + (gg + 1) * _GRP, _GRP)],
                        dst_g[1 - gb], dsem)

                for k in range(_GRP):
                    b = k % 2
                    jj = gg * _GRP + k
                    pltpu.make_async_copy(hp.at[src_v.at[b]], rows_v.at[b],
                                          gsems[b]).wait()
                    pltpu.sync_copy(rows_v.at[b], acc_sh.at[dst_g[gb].at[k]],
                                    add=True)

                    @pl.when(jj + 2 < _CPS)
                    def _():
                        pltpu.async_copy(hp.at[src_v.at[jj + 2]], rows_v.at[b],
                                         gsems[b])

                @pl.when(gg + 1 < _NGRP)
                def _():
                    pltpu.make_async_copy(
                        dstp.at[pl.ds(s * _CPS, _GRP)], dst_g[1 - gb],
                        dsem).wait()

        plsc.subcore_barrier()
        pltpu.sync_copy(acc_sh.at[pl.ds(s * _RPT, _RPT)],
                        out.at[pl.ds(s * _RPT, _RPT)])

    @pl.when(c == 0)
    def _():
        run(hp0, o0)

    @pl.when(c == 1)
    def _():
        run(hp1, o1)


@functools.cache
def _sc_kernels():
    mesh = plsc.VectorSubcoreMesh(core_axis_name="c", subcore_axis_name="s",
                                  num_cores=2, num_subcores=16)
    hshape = jax.ShapeDtypeStruct((_NP, _H), jnp.float32)
    deg = pl.kernel(
        _deg_body,
        out_type=hshape,
        mesh=mesh,
        scratch_types=[
            pltpu.VMEM((_DCPT, _CHUNK), jnp.int32),
            pltpu.VMEM((_NP,), jnp.float32),
            pltpu.VMEM((16, _RPT), jnp.float32),
            pltpu.VMEM((16, _H), jnp.float32),
            pltpu.VMEM_SHARED((16, 16, _RPT), jnp.float32),
        ],
        compiler_params=pltpu.CompilerParams(needs_layout_passes=False),
    )
    spmm = pl.kernel(
        _spmm_body,
        out_type=(hshape, hshape),
        mesh=mesh,
        scratch_types=[
            pltpu.VMEM((_CPS, _CHUNK), jnp.int32),
            [pltpu.VMEM((_GRP, _CHUNK), jnp.int32)] * 2,
            pltpu.VMEM((2, _CHUNK, _H), jnp.float32),
            pltpu.VMEM_SHARED((_NP, _H), jnp.float32),
            [pltpu.SemaphoreType.DMA] * 2,
            pltpu.SemaphoreType.DMA,
        ],
        compiler_params=pltpu.CompilerParams(needs_layout_passes=False),
    )
    return deg, spmm


def _dinv(p_ref):
    return lax.rsqrt(2.0 + p_ref[...])


def _l1_body(x_ref, w_ref, p_ref, o0_ref, o1_ref):
    dinv = _dinv(p_ref)
    hp = jnp.dot(x_ref[...], w_ref[...], preferred_element_type=jnp.float32)
    o0_ref[...] = hp[:, :_H] * dinv
    o1_ref[...] = hp[:, _H:] * dinv


def _mid_body(a0_ref, a1_ref, h0_ref, h1_ref, p_ref, w_ref, b_ref,
              o0_ref, o1_ref):
    dinv = _dinv(p_ref)
    z0 = jnp.maximum(dinv * (a0_ref[...] + 2.0 * h0_ref[...]) + b_ref[0:1, :], 0.0)
    z1 = jnp.maximum(dinv * (a1_ref[...] + 2.0 * h1_ref[...]) + b_ref[1:2, :], 0.0)
    x = jnp.concatenate([z0, z1], axis=1)
    hp = jnp.dot(x, w_ref[...], preferred_element_type=jnp.float32)
    o0_ref[...] = hp[:, :_H] * dinv
    o1_ref[...] = hp[:, _H:] * dinv


def _fin_body(a0_ref, a1_ref, h0_ref, h1_ref, p_ref, b_ref, o_ref):
    dinv = _dinv(p_ref)
    o_ref[:, :_H] = jnp.maximum(
        dinv * (a0_ref[...] + 2.0 * h0_ref[...]) + b_ref[0:1, :], 0.0)
    o_ref[:, _H:] = jnp.maximum(
        dinv * (a1_ref[...] + 2.0 * h1_ref[...]) + b_ref[1:2, :], 0.0)


_half_spec = pl.BlockSpec((_BM, _H), lambda i: (i, 0))
_w_spec = pl.BlockSpec((_D, _D), lambda i: (0, 0))
_b_spec = pl.BlockSpec((2, _H), lambda i: (0, 0))
_half_shape = jax.ShapeDtypeStruct((_NP, _H), jnp.float32)

_l1_tc = pl.pallas_call(
    _l1_body,
    grid=(_NP // _BM,),
    in_specs=[pl.BlockSpec((_BM, _D), lambda i: (i, 0)), _w_spec, _half_spec],
    out_specs=(_half_spec, _half_spec),
    out_shape=(_half_shape, _half_shape),
)

_mid_tc = pl.pallas_call(
    _mid_body,
    grid=(_NP // _BM,),
    in_specs=[_half_spec, _half_spec, _half_spec, _half_spec,
              _half_spec, _w_spec, _b_spec],
    out_specs=(_half_spec, _half_spec),
    out_shape=(_half_shape, _half_shape),
)

_fin_tc = pl.pallas_call(
    _fin_body,
    grid=(_NP // _BM,),
    in_specs=[_half_spec, _half_spec, _half_spec, _half_spec,
              _half_spec, _b_spec],
    out_specs=pl.BlockSpec((_BM, _D), lambda i: (i, 0)),
    out_shape=jax.ShapeDtypeStruct((_NP, _D), jnp.float32),
)


def kernel(nodes_encodings, edge_index, W0, b0, W1, b1, W2, b2):
    deg_sc, spmm_sc = _sc_kernels()
    x = jnp.zeros((_NP, _D), jnp.float32).at[:_N].set(nodes_encodings)
    ei = edge_index.astype(jnp.int32)
    pad = _N + (jnp.arange(_EP - _E, dtype=jnp.int32) % (_NP - _N))
    srcp = jnp.concatenate([ei[0], pad]).reshape(_NCH, _CHUNK)
    dstp = jnp.concatenate([ei[1], pad]).reshape(_NCH, _CHUNK)
    zcol = jnp.zeros((_NP,), jnp.float32)

    p = deg_sc(dstp, zcol)
    h0, h1 = _l1_tc(x, W0, p)
    a0, a1 = spmm_sc(h0, h1, srcp, dstp)
    h0, h1 = _mid_tc(a0, a1, h0, h1, p, W1, b0.reshape(2, _H))
    a0, a1 = spmm_sc(h0, h1, srcp, dstp)
    h0, h1 = _mid_tc(a0, a1, h0, h1, p, W2, b1.reshape(2, _H))
    a0, a1 = spmm_sc(h0, h1, srcp, dstp)
    out = _fin_tc(a0, a1, h0, h1, p, b2.reshape(2, _H))
    return out[:_N]


# speedup vs baseline: 18.2728x; 1.0142x over previous
import functools

import jax
import jax.numpy as jnp
from jax import lax
from jax.experimental import pallas as pl
from jax.experimental.pallas import tpu as pltpu
from jax.experimental.pallas import tpu_sc as plsc

_N = 10000
_NP = 10240
_D = 256
_H = 128
_E = 160000
_EP = 163840
_CHUNK = 128
_NCH = _EP // _CHUNK
_CPS = _NCH // 16
_GRP = 8
_NGRP = _CPS // _GRP
_DCPT = _NCH // 16
_RPT = _NP // 16
_BM = 5120


def _splat(v, i):
    return lax.gather(
        v, jnp.full((16, 1), i, jnp.int32),
        lax.GatherDimensionNumbers(offset_dims=(), collapsed_slice_dims=(0,),
                                   start_index_map=(0,)),
        slice_sizes=(1,), mode=lax.GatherScatterMode.PROMISE_IN_BOUNDS)


def _deg_body(dstp, zcol, p0, dst_v, degl, part_v, rep_v, sh):
    c = lax.axis_index("c")
    s = lax.axis_index("s")
    base = s * _DCPT
    pltpu.sync_copy(dstp.at[pl.ds(base, _DCPT)], dst_v)
    pltpu.sync_copy(zcol, degl)
    ones16 = jnp.full((16,), 1.0, jnp.float32)

    @pl.loop(0, _DCPT)
    def _(j):
        for q in range(_CHUNK // 16):
            iv = dst_v[j, pl.ds(q * 16, 16)]
            plsc.addupdate_scatter(degl, [iv], ones16)

    for seg in range(16):
        pltpu.sync_copy(degl.at[pl.ds(seg * _RPT, _RPT)], sh.at[s, seg])
    plsc.subcore_barrier()

    for t in range(16):
        pltpu.sync_copy(sh.at[t, s], part_v.at[t])

    def emit(p):
        @pl.loop(0, _RPT // 16)
        def _(g):
            v = part_v[0, pl.ds(g * 16, 16)]
            for t in range(1, 16):
                v = v + part_v[t, pl.ds(g * 16, 16)]
            for i in range(16):
                row = _splat(v, i)
                for l in range(_H // 16):
                    rep_v[i, pl.ds(l * 16, 16)] = row
            pltpu.sync_copy(rep_v, p.at[pl.ds(s * _RPT + g * 16, 16)])

    @pl.when(c == 0)
    def _():
        emit(p0)


def _spmm_body(hp0, hp1, srcp, dstp, o0, o1,
               src_v, dst_g, rows_v, acc_sh, gsems, dsem):
    c = lax.axis_index("c")
    s = lax.axis_index("s")
    pltpu.sync_copy(srcp.at[pl.ds(s * _CPS, _CPS)], src_v)

    z16 = jnp.zeros((16,), jnp.float32)

    @pl.loop(0, _CHUNK)
    def _(r):
        for l in range(_H // 16):
            rows_v[0, r, pl.ds(l * 16, 16)] = z16

    for q in range(_RPT // _CHUNK):
        pltpu.sync_copy(rows_v.at[0],
                        acc_sh.at[pl.ds(s * _RPT + q * _CHUNK, _CHUNK)])

    def run(hp, out):
        for b in range(2):
            pltpu.async_copy(hp.at[src_v.at[b]], rows_v.at[b], gsems[b])
        pltpu.sync_copy(dstp.at[pl.ds(s * _CPS, _GRP)], dst_g[0])
        plsc.subcore_barrier()

        @pl.loop(0, _NGRP, step=2)
        def _(g):
            for gb in range(2):
                gg = g + gb

                @pl.when(gg + 1 < _NGRP)
                def _():
                    pltpu.async_copy(
                        dstp.at[pl.ds(s * _CPS + (gg + 1) * _GRP, _GRP)],
                        dst_g[1 - gb], dsem)

                for k in range(_GRP):
                    b = k % 2
                    jj = gg * _GRP + k
                    pltpu.make_async_copy(hp.at[src_v.at[b]], rows_v.at[b],
                                          gsems[b]).wait()
                    pltpu.sync_copy(rows_v.at[b], acc_sh.at[dst_g[gb].at[k]],
                                    add=True)

                    @pl.when(jj + 2 < _CPS)
                    def _():
                        pltpu.async_copy(hp.at[src_v.at[jj + 2]], rows_v.at[b],
                                         gsems[b])

                @pl.when(gg + 1 < _NGRP)
                def _():
                    pltpu.make_async_copy(
                        dstp.at[pl.ds(s * _CPS, _GRP)], dst_g[1 - gb],
                        dsem).wait()

        plsc.subcore_barrier()
        pltpu.sync_copy(acc_sh.at[pl.ds(s * _RPT, _RPT)],
                        out.at[pl.ds(s * _RPT, _RPT)])

    @pl.when(c == 0)
    def _():
        run(hp0, o0)

    @pl.when(c == 1)
    def _():
        run(hp1, o1)


@functools.cache
def _sc_kernels():
    mesh = plsc.VectorSubcoreMesh(core_axis_name="c", subcore_axis_name="s",
                                  num_cores=2, num_subcores=16)
    hshape = jax.ShapeDtypeStruct((_NP, _H), jnp.float32)
    deg = pl.kernel(
        _deg_body,
        out_type=hshape,
        mesh=mesh,
        scratch_types=[
            pltpu.VMEM((_DCPT, _CHUNK), jnp.int32),
            pltpu.VMEM((_NP,), jnp.float32),
            pltpu.VMEM((16, _RPT), jnp.float32),
            pltpu.VMEM((16, _H), jnp.float32),
            pltpu.VMEM_SHARED((16, 16, _RPT), jnp.float32),
        ],
        compiler_params=pltpu.CompilerParams(needs_layout_passes=False),
    )
    spmm = pl.kernel(
        _spmm_body,
        out_type=(hshape, hshape),
        mesh=mesh,
        scratch_types=[
            pltpu.VMEM((_CPS, _CHUNK), jnp.int32),
            [pltpu.VMEM((_GRP, _CHUNK), jnp.int32)] * 2,
            pltpu.VMEM((2, _CHUNK, _H), jnp.float32),
            pltpu.VMEM_SHARED((_NP, _H), jnp.float32),
            [pltpu.SemaphoreType.DMA] * 2,
            pltpu.SemaphoreType.DMA,
        ],
        compiler_params=pltpu.CompilerParams(needs_layout_passes=False),
    )
    return deg, spmm


def _dinv(p_ref):
    return lax.rsqrt(2.0 + p_ref[...])


def _l1_body(x_ref, w_ref, p_ref, o0_ref, o1_ref):
    dinv = _dinv(p_ref)
    hp = jnp.dot(x_ref[...], w_ref[...], preferred_element_type=jnp.float32)
    o0_ref[...] = hp[:, :_H] * dinv
    o1_ref[...] = hp[:, _H:] * dinv


def _mid_body(a0_ref, a1_ref, h0_ref, h1_ref, p_ref, w_ref, b_ref,
              o0_ref, o1_ref):
    dinv = _dinv(p_ref)
    z0 = jnp.maximum(dinv * (a0_ref[...] + 2.0 * h0_ref[...]) + b_ref[0:1, :], 0.0)
    z1 = jnp.maximum(dinv * (a1_ref[...] + 2.0 * h1_ref[...]) + b_ref[1:2, :], 0.0)
    x = jnp.concatenate([z0, z1], axis=1)
    hp = jnp.dot(x, w_ref[...], preferred_element_type=jnp.float32)
    o0_ref[...] = hp[:, :_H] * dinv
    o1_ref[...] = hp[:, _H:] * dinv


def _fin_body(a0_ref, a1_ref, h0_ref, h1_ref, p_ref, b_ref, o_ref):
    dinv = _dinv(p_ref)
    o_ref[:, :_H] = jnp.maximum(
        dinv * (a0_ref[...] + 2.0 * h0_ref[...]) + b_ref[0:1, :], 0.0)
    o_ref[:, _H:] = jnp.maximum(
        dinv * (a1_ref[...] + 2.0 * h1_ref[...]) + b_ref[1:2, :], 0.0)


_half_spec = pl.BlockSpec((_BM, _H), lambda i: (i, 0))
_w_spec = pl.BlockSpec((_D, _D), lambda i: (0, 0))
_b_spec = pl.BlockSpec((2, _H), lambda i: (0, 0))
_half_shape = jax.ShapeDtypeStruct((_NP, _H), jnp.float32)

_l1_tc = pl.pallas_call(
    _l1_body,
    grid=(_NP // _BM,),
    in_specs=[pl.BlockSpec((_BM, _D), lambda i: (i, 0)), _w_spec, _half_spec],
    out_specs=(_half_spec, _half_spec),
    out_shape=(_half_shape, _half_shape),
)

_mid_tc = pl.pallas_call(
    _mid_body,
    grid=(_NP // _BM,),
    in_specs=[_half_spec, _half_spec, _half_spec, _half_spec,
              _half_spec, _w_spec, _b_spec],
    out_specs=(_half_spec, _half_spec),
    out_shape=(_half_shape, _half_shape),
)

_fin_tc = pl.pallas_call(
    _fin_body,
    grid=(_NP // _BM,),
    in_specs=[_half_spec, _half_spec, _half_spec, _half_spec,
              _half_spec, _b_spec],
    out_specs=pl.BlockSpec((_BM, _D), lambda i: (i, 0)),
    out_shape=jax.ShapeDtypeStruct((_NP, _D), jnp.float32),
)


def kernel(nodes_encodings, edge_index, W0, b0, W1, b1, W2, b2):
    deg_sc, spmm_sc = _sc_kernels()
    x = jnp.zeros((_NP, _D), jnp.float32).at[:_N].set(nodes_encodings)
    ei = edge_index.astype(jnp.int32)
    pad = _N + (jnp.arange(_EP - _E, dtype=jnp.int32) % (_NP - _N))
    srcp = jnp.concatenate([ei[0], pad]).reshape(_NCH, _CHUNK)
    dstp = jnp.concatenate([ei[1], pad]).reshape(_NCH, _CHUNK)
    zcol = jnp.zeros((_NP,), jnp.float32)

    p = deg_sc(dstp, zcol)
    h0, h1 = _l1_tc(x, W0, p)
    a0, a1 = spmm_sc(h0, h1, srcp, dstp)
    h0, h1 = _mid_tc(a0, a1, h0, h1, p, W1, b0.reshape(2, _H))
    a0, a1 = spmm_sc(h0, h1, srcp, dstp)
    h0, h1 = _mid_tc(a0, a1, h0, h1, p, W2, b1.reshape(2, _H))
    a0, a1 = spmm_sc(h0, h1, srcp, dstp)
    out = _fin_tc(a0, a1, h0, h1, p, b2.reshape(2, _H))
    return out[:_N]
